# Initial kernel scaffold; baseline (speedup 1.0000x reference)
import jax
import jax.numpy as jnp
from jax.experimental import pallas as pl


def kernel(x, W):
    raise NotImplementedError("write your pallas kernel here")


# speedup vs baseline: 1.9439x; 1.9439x over previous
import functools

import jax
import jax.numpy as jnp
from jax import lax
from jax.experimental import pallas as pl
from jax.experimental.pallas import tpu as pltpu
from jax.experimental.pallas import tpu_sc as plsc

B = 131072
D = 64
HMASK = 524288 - 1
P0, P1, P2 = 73856093, 19349663, 83492791
L = 16
C = 128
NW = 32
PTS_PER_W = B // NW
N_CHUNKS = PTS_PER_W // C


def _i32(v):
    return jnp.asarray(v, jnp.int32)


def _grid_body(x0h, x1h, x2h, W, out, xv, idxv, wv, rowsv, outv, sem):
    wid = (lax.axis_index("s").astype(jnp.int32) * _i32(2)
           + lax.axis_index("c").astype(jnp.int32))
    base0 = wid * _i32(PTS_PER_W)

    def chunk(g, carry):
        base = base0 + g * _i32(C)
        for d, xh in enumerate((x0h, x1h, x2h)):
            pltpu.sync_copy(xh.at[pl.ds(base, C)], xv.at[d])

        def vec(v, carry2):
            off = v * _i32(L)
            sl = pl.ds(off, L)
            x0 = xv[0, sl]
            x1 = xv[1, sl]
            x2 = xv[2, sl]
            xx0 = (x0 + 1.0) * 64.0
            xx1 = (x1 + 1.0) * 64.0
            xx2 = (x2 + 1.0) * 64.0
            i0 = xx0.astype(jnp.int32)
            i1 = xx1.astype(jnp.int32)
            i2 = xx2.astype(jnp.int32)
            f0 = xx0 - i0.astype(jnp.float32)
            f1 = xx1 - i1.astype(jnp.float32)
            f2 = xx2 - i2.astype(jnp.float32)
            m0 = (i0 * _i32(P0), i0 * _i32(P0) + _i32(P0))
            m1 = (i1 * _i32(P1), i1 * _i32(P1) + _i32(P1))
            m2 = (i2 * _i32(P2), i2 * _i32(P2) + _i32(P2))
            w0 = (1.0 - f0, f0)
            w1 = (1.0 - f1, f1)
            w2 = (1.0 - f2, f2)
            pts = off + lax.iota(jnp.int32, L)
            for c in range(8):
                b0, b1, b2 = c & 1, (c >> 1) & 1, (c >> 2) & 1
                idxv[c, sl] = (m0[b0] ^ m1[b1] ^ m2[b2]) & _i32(HMASK)
                plsc.store_scatter(
                    wv, [pts, jnp.full((L,), c, jnp.int32)],
                    w0[b0] * w1[b1] * w2[b2])
            return carry2

        lax.fori_loop(_i32(0), _i32(C // L), vec, _i32(0))

        copies = [pltpu.async_copy(W.at[idxv.at[c]], rowsv.at[c], sem)
                  for c in range(8)]
        for cp in copies:
            cp.wait()

        def acc(p, carry2):
            wrow = wv[p, :]
            ws = [wrow[c] for c in range(8)]
            for j in range(D // L):
                sl = pl.ds(j * L, L)
                s = ws[0] * rowsv[0, p, sl]
                for c in range(1, 8):
                    s = s + ws[c] * rowsv[c, p, sl]
                outv[p, sl] = s
            return carry2

        lax.fori_loop(_i32(0), _i32(C), acc, _i32(0))
        pltpu.sync_copy(outv, out.at[pl.ds(base, C)])
        return carry

    lax.fori_loop(_i32(0), _i32(N_CHUNKS), chunk, _i32(0))


@jax.jit
def kernel(x, W):
    from jax._src import config as _jax_config
    with _jax_config.enable_x64(False):
        return _kernel_impl(x, W)


def _kernel_impl(x, W):
    x0, x1, x2 = x[:, 0], x[:, 1], x[:, 2]
    mesh = plsc.VectorSubcoreMesh(core_axis_name="c", subcore_axis_name="s")
    f = pl.kernel(
        _grid_body,
        out_type=jax.ShapeDtypeStruct((B, D), jnp.float32),
        mesh=mesh,
        scratch_types=[
            pltpu.VMEM((3, C), jnp.float32),
            pltpu.VMEM((8, C), jnp.int32),
            pltpu.VMEM((C, L), jnp.float32),
            pltpu.VMEM((8, C, D), jnp.float32),
            pltpu.VMEM((C, D), jnp.float32),
            pltpu.SemaphoreType.DMA,
        ],
        compiler_params=pltpu.CompilerParams(
            needs_layout_passes=False, use_tc_tiling_on_sc=False),
    )
    return f(x0, x1, x2, W)
